# Initial kernel scaffold; baseline (speedup 1.0000x reference)
import jax
import jax.numpy as jnp
from jax.experimental import pallas as pl


def kernel(x, edge_index, W, b):
    raise NotImplementedError("write your pallas kernel here")


# speedup vs baseline: 11.8613x; 11.8613x over previous
import dataclasses
import functools

import jax
import jax.numpy as jnp
from jax import lax
from jax.experimental import pallas as pl
from jax.experimental.pallas import tpu as pltpu
from jax.experimental.pallas import tpu_sc as plsc

N = 10000
D = 128
BLK = 128
NC, NS = 2, 16
NW = NC * NS
BPW = 80
NBLK = NW * BPW
E_PAD = NBLK * BLK
ROWS = 10240
RPS = ROWS // NS
DUMMY = N

_mesh = plsc.VectorSubcoreMesh(
    core_axis_name="c", subcore_axis_name="s", num_cores=NC, num_subcores=NS
)

_sc_params = pltpu.CompilerParams()
if "needs_layout_passes" in pltpu.CompilerParams.__dataclass_fields__:
    _sc_params = dataclasses.replace(_sc_params, needs_layout_passes=False)


@functools.partial(
    pl.kernel,
    out_type=jax.ShapeDtypeStruct((NW, ROWS), jnp.float32),
    mesh=_mesh,
    scratch_types=[
        pltpu.VMEM((ROWS,), jnp.float32),
        pltpu.VMEM((BPW, BLK), jnp.int32),
    ],
    compiler_params=_sc_params,
)
def _sc_degree(dst_hbm, out_hbm, hist_v, idx_v):
    c = lax.axis_index("c")
    s = lax.axis_index("s")
    w = c * NS + s

    @pl.loop(0, ROWS, step=16)
    def _zero(i):
        hist_v[pl.ds(i, 16)] = jnp.zeros((16,), jnp.float32)

    pltpu.sync_copy(dst_hbm.at[pl.ds(w * BPW, BPW)], idx_v)
    ones = jnp.ones((16,), jnp.float32)

    @pl.loop(0, BPW)
    def _blk(j):
        @pl.loop(0, BLK, step=16)
        def _grp(k):
            idx = idx_v[j, pl.ds(k, 16)]
            plsc.addupdate_scatter(hist_v, [idx], ones)

    pltpu.sync_copy(hist_v, out_hbm.at[w])


@functools.partial(
    pl.kernel,
    out_type=jax.ShapeDtypeStruct((NC, ROWS, D), jnp.float32),
    mesh=_mesh,
    scratch_types=[
        pltpu.VMEM((BPW, BLK), jnp.int32),
        pltpu.VMEM((BPW, BLK), jnp.int32),
        pltpu.VMEM((BLK, D), jnp.float32),
        pltpu.VMEM_SHARED((ROWS, D), jnp.float32),
        pltpu.SemaphoreType.DMA,
    ],
)
def _sc_aggregate(g_hbm, src_hbm, dst_hbm, out_hbm,
                  si_v, di_v, rows_v, acc_sh, sem):
    c = lax.axis_index("c")
    s = lax.axis_index("s")
    w = c * NS + s

    @pl.loop(0, BLK)
    def _z0(i):
        @pl.loop(0, D, step=16)
        def _z1(k):
            rows_v[i, pl.ds(k, 16)] = jnp.zeros((16,), jnp.float32)

    @pl.loop(0, RPS, step=BLK)
    def _z2(r):
        pltpu.sync_copy(rows_v, acc_sh.at[pl.ds(s * RPS + r, BLK)])

    pltpu.sync_copy(src_hbm.at[pl.ds(w * BPW, BPW)], si_v)
    pltpu.sync_copy(dst_hbm.at[pl.ds(w * BPW, BPW)], di_v)

    plsc.subcore_barrier()

    @pl.loop(0, BPW)
    def _edge(j):
        pltpu.async_copy(g_hbm.at[si_v.at[j]], rows_v, sem).wait()
        pltpu.sync_copy(rows_v, acc_sh.at[di_v.at[j]], add=True)

    plsc.subcore_barrier()

    pltpu.sync_copy(
        acc_sh.at[pl.ds(s * RPS, RPS)],
        out_hbm.at[c, pl.ds(s * RPS, RPS)],
    )


def _mm_body(x_ref, w_ref, o_ref):
    o_ref[...] = lax.dot_general(
        x_ref[...], w_ref[...], (((1,), (1,)), ((), ())),
        preferred_element_type=jnp.float32,
        precision=lax.Precision.HIGHEST,
    )


def _tc_linear(x, W):
    R = 2000
    return pl.pallas_call(
        _mm_body,
        grid=(N // R,),
        in_specs=[
            pl.BlockSpec((R, D), lambda i: (i, 0)),
            pl.BlockSpec((D, D), lambda i: (0, 0)),
        ],
        out_specs=pl.BlockSpec((R, D), lambda i: (i, 0)),
        out_shape=jax.ShapeDtypeStruct((N, D), jnp.float32),
    )(x, W)


def _dis_body(hist_ref, o_ref):
    deg = jnp.sum(hist_ref[...], axis=0) + 1.0
    o_ref[...] = lax.rsqrt(deg)


def _tc_dis(hist):
    return pl.pallas_call(
        _dis_body,
        out_shape=jax.ShapeDtypeStruct((ROWS,), jnp.float32),
    )(hist)


def _scale_body(h_ref, d_ref, o_ref):
    o_ref[...] = h_ref[...] * d_ref[...]


def _tc_scale(h, dis_col):
    R = 2000
    return pl.pallas_call(
        _scale_body,
        grid=(N // R,),
        in_specs=[
            pl.BlockSpec((R, D), lambda i: (i, 0)),
            pl.BlockSpec((R, 1), lambda i: (i, 0)),
        ],
        out_specs=pl.BlockSpec((R, D), lambda i: (i, 0)),
        out_shape=jax.ShapeDtypeStruct((N, D), jnp.float32),
    )(h, dis_col)


def _epi_body(p0_ref, p1_ref, g_ref, d_ref, b_ref, o_ref):
    o_ref[...] = d_ref[...] * (p0_ref[...] + p1_ref[...] + g_ref[...]) + b_ref[...]


def _tc_epilogue(p0, p1, g, dis_col, b_row):
    R = 2000
    part_spec = pl.BlockSpec((R, D), lambda i: (i, 0))
    return pl.pallas_call(
        _epi_body,
        grid=(N // R,),
        in_specs=[
            part_spec,
            part_spec,
            pl.BlockSpec((R, D), lambda i: (i, 0)),
            pl.BlockSpec((R, 1), lambda i: (i, 0)),
            pl.BlockSpec((1, D), lambda i: (0, 0)),
        ],
        out_specs=pl.BlockSpec((R, D), lambda i: (i, 0)),
        out_shape=jax.ShapeDtypeStruct((N, D), jnp.float32),
    )(p0, p1, g, dis_col, b_row)


def kernel(x, edge_index, W, b):
    e = edge_index.shape[1]
    src = edge_index[0].astype(jnp.int32)
    dst = edge_index[1].astype(jnp.int32)
    pad = E_PAD - e
    src_p = jnp.concatenate([src, jnp.zeros((pad,), jnp.int32)]).reshape(NBLK, BLK)
    dst_p = jnp.concatenate([dst, jnp.full((pad,), DUMMY, jnp.int32)]).reshape(NBLK, BLK)

    h = _tc_linear(x, W)
    hist = _sc_degree(dst_p)
    dis = _tc_dis(hist)
    dis_col = dis[:N].reshape(N, 1)
    g = _tc_scale(h, dis_col)
    parts = _sc_aggregate(g, src_p, dst_p)
    out = _tc_epilogue(parts[0], parts[1], g, dis_col, b.reshape(1, D))
    return out


# speedup vs baseline: 30.8781x; 2.6033x over previous
import dataclasses
import functools

import jax
import jax.numpy as jnp
from jax import lax
from jax.experimental import pallas as pl
from jax.experimental.pallas import tpu as pltpu
from jax.experimental.pallas import tpu_sc as plsc

N = 10000
D = 128
BLK = 128
NC, NS = 2, 16
NW = NC * NS
BPW = 80
NBLK = NW * BPW
E_PAD = NBLK * BLK
ROWS = 10240
RPS = ROWS // NS
DUMMY = N
TBLK = 64
TPW = E_PAD // (NW * TBLK)
NTR = E_PAD // TBLK

_mesh = plsc.VectorSubcoreMesh(
    core_axis_name="c", subcore_axis_name="s", num_cores=NC, num_subcores=NS
)

_sc_params = pltpu.CompilerParams()
if "needs_layout_passes" in pltpu.CompilerParams.__dataclass_fields__:
    _sc_params = dataclasses.replace(_sc_params, needs_layout_passes=False)


@functools.partial(
    pl.kernel,
    out_type=jax.ShapeDtypeStruct((NW, ROWS), jnp.float32),
    mesh=_mesh,
    scratch_types=[
        pltpu.VMEM((ROWS,), jnp.float32),
        pltpu.VMEM((TPW, TBLK), jnp.int32),
    ],
    compiler_params=_sc_params,
)
def _sc_degree(dst_hbm, out_hbm, hist_v, idx_v):
    c = lax.axis_index("c")
    s = lax.axis_index("s")
    w = c * NS + s

    @pl.loop(0, ROWS, step=16)
    def _zero(i):
        hist_v[pl.ds(i, 16)] = jnp.zeros((16,), jnp.float32)

    pltpu.sync_copy(dst_hbm.at[pl.ds(w * TPW, TPW)], idx_v)
    ones = jnp.ones((16,), jnp.float32)

    @pl.loop(0, TPW)
    def _blk(j):
        @pl.loop(0, TBLK, step=16)
        def _grp(k):
            idx = idx_v[j, pl.ds(k, 16)]
            plsc.addupdate_scatter(hist_v, [idx], ones)

    pltpu.sync_copy(hist_v, out_hbm.at[w])


@functools.partial(
    pl.kernel,
    out_type=jax.ShapeDtypeStruct((NC, ROWS, D), jnp.float32),
    mesh=_mesh,
    scratch_types=[
        pltpu.VMEM((TPW // 2, TBLK), jnp.int32),
        pltpu.VMEM((TPW // 2, TBLK), jnp.int32),
        pltpu.VMEM((TBLK, D), jnp.float32),
        pltpu.VMEM((TBLK, D), jnp.float32),
        pltpu.VMEM_SHARED((ROWS, D), jnp.float32),
        pltpu.SemaphoreType.DMA,
    ],
)
def _sc_aggregate(g_hbm, src_hbm, dst_hbm, out_hbm,
                  si_v, di_v, rows0_v, rows1_v, acc_sh, sem):
    c = lax.axis_index("c")
    s = lax.axis_index("s")
    w = c * NS + s
    hpw = TPW // 2

    @pl.loop(0, TBLK)
    def _z0(i):
        @pl.loop(0, D, step=16)
        def _z1(k):
            rows0_v[i, pl.ds(k, 16)] = jnp.zeros((16,), jnp.float32)

    @pl.loop(0, RPS, step=TBLK)
    def _z2(r):
        pltpu.sync_copy(rows0_v, acc_sh.at[pl.ds(s * RPS + r, TBLK)])

    plsc.subcore_barrier()

    for ph in range(2):
        base = w * TPW + ph * hpw
        pltpu.sync_copy(src_hbm.at[pl.ds(base, hpw)], si_v)
        pltpu.sync_copy(dst_hbm.at[pl.ds(base, hpw)], di_v)

        pltpu.async_copy(g_hbm.at[si_v.at[0]], rows0_v, sem)

        @pl.loop(0, hpw, step=2)
        def _edge(j):
            pltpu.make_async_copy(g_hbm.at[si_v.at[j]], rows0_v, sem).wait()
            pltpu.async_copy(g_hbm.at[si_v.at[j + 1]], rows1_v, sem)
            pltpu.sync_copy(rows0_v, acc_sh.at[di_v.at[j]], add=True)

            pltpu.make_async_copy(g_hbm.at[si_v.at[j + 1]], rows1_v, sem).wait()

            @pl.when(j + 2 < hpw)
            def _next():
                pltpu.async_copy(g_hbm.at[si_v.at[j + 2]], rows0_v, sem)

            pltpu.sync_copy(rows1_v, acc_sh.at[di_v.at[j + 1]], add=True)

    plsc.subcore_barrier()

    pltpu.sync_copy(
        acc_sh.at[pl.ds(s * RPS, RPS)],
        out_hbm.at[c, pl.ds(s * RPS, RPS)],
    )


def _mm_body(x_ref, w_ref, o_ref):
    o_ref[...] = lax.dot_general(
        x_ref[...], w_ref[...], (((1,), (1,)), ((), ())),
        preferred_element_type=jnp.float32,
        precision=lax.Precision.HIGHEST,
    )


def _tc_linear(x, W):
    R = 2000
    return pl.pallas_call(
        _mm_body,
        grid=(N // R,),
        in_specs=[
            pl.BlockSpec((R, D), lambda i: (i, 0)),
            pl.BlockSpec((D, D), lambda i: (0, 0)),
        ],
        out_specs=pl.BlockSpec((R, D), lambda i: (i, 0)),
        out_shape=jax.ShapeDtypeStruct((N, D), jnp.float32),
    )(x, W)


def _dis_body(hist_ref, o_ref):
    deg = jnp.sum(hist_ref[...], axis=0) + 1.0
    o_ref[...] = lax.rsqrt(deg)


def _tc_dis(hist):
    return pl.pallas_call(
        _dis_body,
        out_shape=jax.ShapeDtypeStruct((ROWS,), jnp.float32),
    )(hist)


def _scale_body(h_ref, d_ref, o_ref):
    o_ref[...] = h_ref[...] * d_ref[...]


def _tc_scale(h, dis_col):
    R = 2000
    return pl.pallas_call(
        _scale_body,
        grid=(N // R,),
        in_specs=[
            pl.BlockSpec((R, D), lambda i: (i, 0)),
            pl.BlockSpec((R, 1), lambda i: (i, 0)),
        ],
        out_specs=pl.BlockSpec((R, D), lambda i: (i, 0)),
        out_shape=jax.ShapeDtypeStruct((N, D), jnp.float32),
    )(h, dis_col)


def _epi_body(p0_ref, p1_ref, g_ref, d_ref, b_ref, o_ref):
    o_ref[...] = d_ref[...] * (p0_ref[...] + p1_ref[...] + g_ref[...]) + b_ref[...]


def _tc_epilogue(p0, p1, g, dis_col, b_row):
    R = 2000
    part_spec = pl.BlockSpec((R, D), lambda i: (i, 0))
    return pl.pallas_call(
        _epi_body,
        grid=(N // R,),
        in_specs=[
            part_spec,
            part_spec,
            pl.BlockSpec((R, D), lambda i: (i, 0)),
            pl.BlockSpec((R, 1), lambda i: (i, 0)),
            pl.BlockSpec((1, D), lambda i: (0, 0)),
        ],
        out_specs=pl.BlockSpec((R, D), lambda i: (i, 0)),
        out_shape=jax.ShapeDtypeStruct((N, D), jnp.float32),
    )(p0, p1, g, dis_col, b_row)


def kernel(x, edge_index, W, b):
    e = edge_index.shape[1]
    src = edge_index[0].astype(jnp.int32)
    dst = edge_index[1].astype(jnp.int32)
    pad = E_PAD - e
    pad_ar = jnp.arange(pad, dtype=jnp.int32)
    src_p = jnp.concatenate([src, pad_ar % N]).reshape(NTR, TBLK)
    dst_p = jnp.concatenate([dst, DUMMY + pad_ar % (ROWS - N)]).reshape(NTR, TBLK)

    h = _tc_linear(x, W)
    hist = _sc_degree(dst_p)
    dis = _tc_dis(hist)
    dis_col = dis[:N].reshape(N, 1)
    g = _tc_scale(h, dis_col)
    parts = _sc_aggregate(g, src_p, dst_p)
    out = _tc_epilogue(parts[0], parts[1], g, dis_col, b.reshape(1, D))
    return out


# speedup vs baseline: 39.6452x; 1.2839x over previous
import dataclasses
import functools

import jax
import jax.numpy as jnp
from jax import lax
from jax.experimental import pallas as pl
from jax.experimental.pallas import tpu as pltpu
from jax.experimental.pallas import tpu_sc as plsc

N = 10000
D = 128
BLK = 128
NC, NS = 2, 16
NW = NC * NS
BPW = 80
NBLK = NW * BPW
E_PAD = NBLK * BLK
ROWS = 10112
RPS = ROWS // NS
DUMMY = N
TBLK = 32
NBUF = 4
TPW = E_PAD // (NW * TBLK)
NTR = E_PAD // TBLK
PHASES = 4
HPW = TPW // PHASES

_mesh = plsc.VectorSubcoreMesh(
    core_axis_name="c", subcore_axis_name="s", num_cores=NC, num_subcores=NS
)

_sc_params = pltpu.CompilerParams()
if "needs_layout_passes" in pltpu.CompilerParams.__dataclass_fields__:
    _sc_params = dataclasses.replace(_sc_params, needs_layout_passes=False)


@functools.partial(
    pl.kernel,
    out_type=jax.ShapeDtypeStruct((NW, ROWS), jnp.float32),
    mesh=_mesh,
    scratch_types=[
        pltpu.VMEM((ROWS,), jnp.float32),
        pltpu.VMEM((TPW, TBLK), jnp.int32),
    ],
    compiler_params=_sc_params,
)
def _sc_degree(dst_hbm, out_hbm, hist_v, idx_v):
    c = lax.axis_index("c")
    s = lax.axis_index("s")
    w = c * NS + s

    @pl.loop(0, ROWS, step=16)
    def _zero(i):
        hist_v[pl.ds(i, 16)] = jnp.zeros((16,), jnp.float32)

    pltpu.sync_copy(dst_hbm.at[pl.ds(w * TPW, TPW)], idx_v)
    ones = jnp.ones((16,), jnp.float32)

    @pl.loop(0, TPW)
    def _blk(j):
        @pl.loop(0, TBLK, step=16)
        def _grp(k):
            idx = idx_v[j, pl.ds(k, 16)]
            plsc.addupdate_scatter(hist_v, [idx], ones)

    pltpu.sync_copy(hist_v, out_hbm.at[w])


@functools.partial(
    pl.kernel,
    out_type=jax.ShapeDtypeStruct((NC, ROWS, D), jnp.float32),
    mesh=_mesh,
    scratch_types=[
        pltpu.VMEM((2 * HPW, TBLK), jnp.int32),
        pltpu.VMEM((NBUF * TBLK, D), jnp.float32),
        pltpu.VMEM_SHARED((ROWS, D), jnp.float32),
        pltpu.SemaphoreType.DMA,
    ],
)
def _sc_aggregate(g_hbm, src_hbm, dst_hbm, out_hbm,
                  idx_v, rows_v, acc_sh, sem):
    c = lax.axis_index("c")
    s = lax.axis_index("s")
    w = c * NS + s
    bufs = tuple(rows_v.at[pl.ds(i * TBLK, TBLK)] for i in range(NBUF))

    @pl.loop(0, NBUF * TBLK)
    def _z0(i):
        @pl.loop(0, D, step=16)
        def _z1(k):
            rows_v[i, pl.ds(k, 16)] = jnp.zeros((16,), jnp.float32)

    zmain = RPS - RPS % (NBUF * TBLK)
    ztail = RPS - zmain

    @pl.loop(0, zmain, step=NBUF * TBLK)
    def _z2(r):
        pltpu.sync_copy(rows_v, acc_sh.at[pl.ds(s * RPS + r, NBUF * TBLK)])

    if ztail:
        pltpu.sync_copy(
            rows_v.at[pl.ds(0, ztail)],
            acc_sh.at[pl.ds(s * RPS + zmain, ztail)],
        )

    plsc.subcore_barrier()

    for ph in range(PHASES):
        base = w * TPW + ph * HPW
        pltpu.sync_copy(src_hbm.at[pl.ds(base, HPW)], idx_v.at[pl.ds(0, HPW)])
        pltpu.sync_copy(dst_hbm.at[pl.ds(base, HPW)], idx_v.at[pl.ds(HPW, HPW)])

        for t in range(NBUF - 1):
            pltpu.async_copy(g_hbm.at[idx_v.at[t]], bufs[t], sem)

        @pl.loop(0, HPW, step=NBUF)
        def _edge(j):
            for i in range(NBUF):
                buf = bufs[i]
                nbuf = bufs[(i + NBUF - 1) % NBUF]
                t = j + i
                pltpu.make_async_copy(g_hbm.at[idx_v.at[t]], buf, sem).wait()

                @pl.when(t + NBUF - 1 < HPW)
                def _next(t=t, nbuf=nbuf):
                    pltpu.async_copy(g_hbm.at[idx_v.at[t + NBUF - 1]], nbuf, sem)

                pltpu.sync_copy(buf, acc_sh.at[idx_v.at[HPW + t]], add=True)

    plsc.subcore_barrier()

    pltpu.sync_copy(
        acc_sh.at[pl.ds(s * RPS, RPS)],
        out_hbm.at[c, pl.ds(s * RPS, RPS)],
    )


def _mm_body(x_ref, w_ref, o_ref):
    o_ref[...] = lax.dot_general(
        x_ref[...], w_ref[...], (((1,), (1,)), ((), ())),
        preferred_element_type=jnp.float32,
        precision=lax.Precision.HIGHEST,
    )


def _tc_linear(x, W):
    R = 2000
    return pl.pallas_call(
        _mm_body,
        grid=(N // R,),
        in_specs=[
            pl.BlockSpec((R, D), lambda i: (i, 0)),
            pl.BlockSpec((D, D), lambda i: (0, 0)),
        ],
        out_specs=pl.BlockSpec((R, D), lambda i: (i, 0)),
        out_shape=jax.ShapeDtypeStruct((N, D), jnp.float32),
    )(x, W)


def _dis_body(hist_ref, o_ref):
    deg = jnp.sum(hist_ref[...], axis=0) + 1.0
    o_ref[...] = lax.rsqrt(deg)


def _tc_dis(hist):
    return pl.pallas_call(
        _dis_body,
        out_shape=jax.ShapeDtypeStruct((ROWS,), jnp.float32),
    )(hist)


def _scale_body(h_ref, d_ref, o_ref):
    o_ref[...] = h_ref[...] * d_ref[...]


def _tc_scale(h, dis_col):
    R = 2000
    return pl.pallas_call(
        _scale_body,
        grid=(N // R,),
        in_specs=[
            pl.BlockSpec((R, D), lambda i: (i, 0)),
            pl.BlockSpec((R, 1), lambda i: (i, 0)),
        ],
        out_specs=pl.BlockSpec((R, D), lambda i: (i, 0)),
        out_shape=jax.ShapeDtypeStruct((N, D), jnp.float32),
    )(h, dis_col)


def _epi_body(p0_ref, p1_ref, g_ref, d_ref, b_ref, o_ref):
    o_ref[...] = d_ref[...] * (p0_ref[...] + p1_ref[...] + g_ref[...]) + b_ref[...]


def _tc_epilogue(p0, p1, g, dis_col, b_row):
    R = 2000
    part_spec = pl.BlockSpec((R, D), lambda i: (i, 0))
    return pl.pallas_call(
        _epi_body,
        grid=(N // R,),
        in_specs=[
            part_spec,
            part_spec,
            pl.BlockSpec((R, D), lambda i: (i, 0)),
            pl.BlockSpec((R, 1), lambda i: (i, 0)),
            pl.BlockSpec((1, D), lambda i: (0, 0)),
        ],
        out_specs=pl.BlockSpec((R, D), lambda i: (i, 0)),
        out_shape=jax.ShapeDtypeStruct((N, D), jnp.float32),
    )(p0, p1, g, dis_col, b_row)


def kernel(x, edge_index, W, b):
    e = edge_index.shape[1]
    src = edge_index[0].astype(jnp.int32)
    dst = edge_index[1].astype(jnp.int32)
    pad = E_PAD - e
    pad_ar = jnp.arange(pad, dtype=jnp.int32)
    src_p = jnp.concatenate([src, pad_ar % N]).reshape(NTR, TBLK)
    dst_p = jnp.concatenate([dst, DUMMY + pad_ar % (ROWS - N)]).reshape(NTR, TBLK)

    h = _tc_linear(x, W)
    hist = _sc_degree(dst_p)
    dis = _tc_dis(hist)
    dis_col = dis[:N].reshape(N, 1)
    g = _tc_scale(h, dis_col)
    parts = _sc_aggregate(g, src_p, dst_p)
    out = _tc_epilogue(parts[0], parts[1], g, dis_col, b.reshape(1, D))
    return out


# speedup vs baseline: 42.2553x; 1.0658x over previous
import dataclasses
import functools

import jax
import jax.numpy as jnp
from jax import lax
from jax.experimental import pallas as pl
from jax.experimental.pallas import tpu as pltpu
from jax.experimental.pallas import tpu_sc as plsc

N = 10000
D = 128
BLK = 128
NC, NS = 2, 16
NW = NC * NS
BPW = 80
NBLK = NW * BPW
E_PAD = NBLK * BLK
ROWS = 10112
RPS = ROWS // NS
DUMMY = N
TBLK = 32
NBUF = 4
TPW = E_PAD // (NW * TBLK)
NTR = E_PAD // TBLK
PHASES = 4
HPW = TPW // PHASES

_mesh = plsc.VectorSubcoreMesh(
    core_axis_name="c", subcore_axis_name="s", num_cores=NC, num_subcores=NS
)

_sc_params = pltpu.CompilerParams()
if "needs_layout_passes" in pltpu.CompilerParams.__dataclass_fields__:
    _sc_params = dataclasses.replace(_sc_params, needs_layout_passes=False)


@functools.partial(
    pl.kernel,
    out_type=jax.ShapeDtypeStruct((NW, ROWS), jnp.float32),
    mesh=_mesh,
    scratch_types=[
        pltpu.VMEM((ROWS,), jnp.float32),
        pltpu.VMEM((TPW, TBLK), jnp.int32),
    ],
    compiler_params=_sc_params,
)
def _sc_degree(dst_hbm, out_hbm, hist_v, idx_v):
    c = lax.axis_index("c")
    s = lax.axis_index("s")
    w = c * NS + s

    @pl.loop(0, ROWS, step=16)
    def _zero(i):
        hist_v[pl.ds(i, 16)] = jnp.zeros((16,), jnp.float32)

    pltpu.sync_copy(dst_hbm.at[pl.ds(w * TPW, TPW)], idx_v)
    ones = jnp.ones((16,), jnp.float32)

    @pl.loop(0, TPW)
    def _blk(j):
        @pl.loop(0, TBLK, step=16)
        def _grp(k):
            idx = idx_v[j, pl.ds(k, 16)]
            plsc.addupdate_scatter(hist_v, [idx], ones)

    pltpu.sync_copy(hist_v, out_hbm.at[w])


@functools.partial(
    pl.kernel,
    out_type=jax.ShapeDtypeStruct((NC, ROWS, D), jnp.float32),
    mesh=_mesh,
    scratch_types=[
        pltpu.VMEM((2 * HPW, TBLK), jnp.int32),
        pltpu.VMEM((NBUF * TBLK, D), jnp.float32),
        pltpu.VMEM_SHARED((ROWS, D), jnp.float32),
        pltpu.SemaphoreType.DMA,
    ],
)
def _sc_aggregate(g_hbm, src_hbm, dst_hbm, out_hbm,
                  idx_v, rows_v, acc_sh, sem):
    c = lax.axis_index("c")
    s = lax.axis_index("s")
    w = c * NS + s
    bufs = tuple(rows_v.at[pl.ds(i * TBLK, TBLK)] for i in range(NBUF))

    @pl.loop(0, NBUF * TBLK)
    def _z0(i):
        @pl.loop(0, D, step=16)
        def _z1(k):
            rows_v[i, pl.ds(k, 16)] = jnp.zeros((16,), jnp.float32)

    zmain = RPS - RPS % (NBUF * TBLK)
    ztail = RPS - zmain

    @pl.loop(0, zmain, step=NBUF * TBLK)
    def _z2(r):
        pltpu.sync_copy(rows_v, acc_sh.at[pl.ds(s * RPS + r, NBUF * TBLK)])

    if ztail:
        pltpu.sync_copy(
            rows_v.at[pl.ds(0, ztail)],
            acc_sh.at[pl.ds(s * RPS + zmain, ztail)],
        )

    plsc.subcore_barrier()

    for ph in range(PHASES):
        base = w * TPW + ph * HPW
        pltpu.sync_copy(src_hbm.at[pl.ds(base, HPW)], idx_v.at[pl.ds(0, HPW)])
        pltpu.sync_copy(dst_hbm.at[pl.ds(base, HPW)], idx_v.at[pl.ds(HPW, HPW)])

        for t in range(NBUF - 1):
            pltpu.async_copy(g_hbm.at[idx_v.at[t]], bufs[t], sem)

        @pl.loop(0, HPW, step=NBUF)
        def _edge(j):
            for i in range(NBUF):
                buf = bufs[i]
                nbuf = bufs[(i + NBUF - 1) % NBUF]
                t = j + i
                pltpu.make_async_copy(g_hbm.at[idx_v.at[t]], buf, sem).wait()

                @pl.when(t + NBUF - 1 < HPW)
                def _next(t=t, nbuf=nbuf):
                    pltpu.async_copy(g_hbm.at[idx_v.at[t + NBUF - 1]], nbuf, sem)

                pltpu.sync_copy(buf, acc_sh.at[idx_v.at[HPW + t]], add=True)

    plsc.subcore_barrier()

    pltpu.sync_copy(
        acc_sh.at[pl.ds(s * RPS, RPS)],
        out_hbm.at[c, pl.ds(s * RPS, RPS)],
    )


def _mm_body(x_ref, w_ref, o_ref):
    o_ref[...] = lax.dot_general(
        x_ref[...], w_ref[...], (((1,), (1,)), ((), ())),
        preferred_element_type=jnp.float32,
        precision=lax.Precision.HIGHEST,
    )


def _tc_linear(x, W):
    R = 2000
    return pl.pallas_call(
        _mm_body,
        grid=(N // R,),
        in_specs=[
            pl.BlockSpec((R, D), lambda i: (i, 0)),
            pl.BlockSpec((D, D), lambda i: (0, 0)),
        ],
        out_specs=pl.BlockSpec((R, D), lambda i: (i, 0)),
        out_shape=jax.ShapeDtypeStruct((N, D), jnp.float32),
    )(x, W)


def _scale_body(hist_ref, h_ref, g_ref, d_ref):
    ones = jnp.ones((NW, 1), jnp.float32)
    deg = lax.dot_general(
        hist_ref[...], ones, (((0,), (0,)), ((), ())),
        preferred_element_type=jnp.float32,
        precision=lax.Precision.HIGHEST,
    ) + 1.0
    dis = lax.rsqrt(deg[:N])
    d_ref[...] = dis
    g_ref[...] = h_ref[...] * dis


def _tc_scale(hist, h):
    return pl.pallas_call(
        _scale_body,
        out_shape=[
            jax.ShapeDtypeStruct((N, D), jnp.float32),
            jax.ShapeDtypeStruct((N, 1), jnp.float32),
        ],
    )(hist, h)


def _epi_body(p_ref0, p_ref1, g_ref, d_ref, b_ref, o_ref):
    p0 = p_ref0[0]
    p1 = p_ref1[0]
    o_ref[...] = d_ref[...] * (p0 + p1 + g_ref[...]) + b_ref[...]


def _tc_epilogue(parts, g, dis_col, b_row):
    R = 2000
    return pl.pallas_call(
        _epi_body,
        grid=(N // R,),
        in_specs=[
            pl.BlockSpec((1, R, D), lambda i: (0, i, 0)),
            pl.BlockSpec((1, R, D), lambda i: (1, i, 0)),
            pl.BlockSpec((R, D), lambda i: (i, 0)),
            pl.BlockSpec((R, 1), lambda i: (i, 0)),
            pl.BlockSpec((1, D), lambda i: (0, 0)),
        ],
        out_specs=pl.BlockSpec((R, D), lambda i: (i, 0)),
        out_shape=jax.ShapeDtypeStruct((N, D), jnp.float32),
    )(parts, parts, g, dis_col, b_row)


def kernel(x, edge_index, W, b):
    e = edge_index.shape[1]
    src = edge_index[0].astype(jnp.int32)
    dst = edge_index[1].astype(jnp.int32)
    pad = E_PAD - e
    pad_ar = jnp.arange(pad, dtype=jnp.int32)
    src_p = jnp.concatenate([src, pad_ar % N]).reshape(NTR, TBLK)
    dst_p = jnp.concatenate([dst, DUMMY + pad_ar % (ROWS - N)]).reshape(NTR, TBLK)

    h = _tc_linear(x, W)
    hist = _sc_degree(dst_p)
    g, dis_col = _tc_scale(hist, h)
    parts = _sc_aggregate(g, src_p, dst_p)
    out = _tc_epilogue(parts, g, dis_col, b.reshape(1, D))
    return out


# speedup vs baseline: 46.1115x; 1.0913x over previous
import dataclasses
import functools

import jax
import jax.numpy as jnp
from jax import lax
from jax.experimental import pallas as pl
from jax.experimental.pallas import tpu as pltpu
from jax.experimental.pallas import tpu_sc as plsc

N = 10000
D = 128
BLK = 128
NC, NS = 2, 16
NW = NC * NS
BPW = 80
NBLK = NW * BPW
E_PAD = NBLK * BLK
ROWS = 10112
RPS = ROWS // NS
DUMMY = N
TBLK = 32
NBUF = 5
TPW = E_PAD // (NW * TBLK)
NTR = E_PAD // TBLK
PHASES = 4
HPW = TPW // PHASES

_mesh = plsc.VectorSubcoreMesh(
    core_axis_name="c", subcore_axis_name="s", num_cores=NC, num_subcores=NS
)

_sc_params = pltpu.CompilerParams()
if "needs_layout_passes" in pltpu.CompilerParams.__dataclass_fields__:
    _sc_params = dataclasses.replace(_sc_params, needs_layout_passes=False)


@functools.partial(
    pl.kernel,
    out_type=jax.ShapeDtypeStruct((NW, ROWS), jnp.float32),
    mesh=_mesh,
    scratch_types=[
        pltpu.VMEM((ROWS,), jnp.float32),
        pltpu.VMEM((TPW, TBLK), jnp.int32),
    ],
    compiler_params=_sc_params,
)
def _sc_degree(dst_hbm, out_hbm, hist_v, idx_v):
    c = lax.axis_index("c")
    s = lax.axis_index("s")
    w = c * NS + s

    @pl.loop(0, ROWS, step=16)
    def _zero(i):
        hist_v[pl.ds(i, 16)] = jnp.zeros((16,), jnp.float32)

    pltpu.sync_copy(dst_hbm.at[pl.ds(w * TPW, TPW)], idx_v)
    ones = jnp.ones((16,), jnp.float32)

    @pl.loop(0, TPW)
    def _blk(j):
        @pl.loop(0, TBLK, step=16)
        def _grp(k):
            idx = idx_v[j, pl.ds(k, 16)]
            plsc.addupdate_scatter(hist_v, [idx], ones)

    pltpu.sync_copy(hist_v, out_hbm.at[w])


@functools.partial(
    pl.kernel,
    out_type=jax.ShapeDtypeStruct((NC, ROWS, D), jnp.float32),
    mesh=_mesh,
    scratch_types=[
        pltpu.VMEM((2 * HPW, TBLK), jnp.int32),
        pltpu.VMEM((NBUF * TBLK, D), jnp.float32),
        pltpu.VMEM_SHARED((ROWS, D), jnp.float32),
        pltpu.SemaphoreType.DMA,
    ],
)
def _sc_aggregate(g_hbm, src_hbm, dst_hbm, out_hbm,
                  idx_v, rows_v, acc_sh, sem):
    c = lax.axis_index("c")
    s = lax.axis_index("s")
    w = c * NS + s
    bufs = tuple(rows_v.at[pl.ds(i * TBLK, TBLK)] for i in range(NBUF))

    @pl.loop(0, NBUF * TBLK)
    def _z0(i):
        @pl.loop(0, D, step=16)
        def _z1(k):
            rows_v[i, pl.ds(k, 16)] = jnp.zeros((16,), jnp.float32)

    zmain = RPS - RPS % (NBUF * TBLK)
    ztail = RPS - zmain

    @pl.loop(0, zmain, step=NBUF * TBLK)
    def _z2(r):
        pltpu.sync_copy(rows_v, acc_sh.at[pl.ds(s * RPS + r, NBUF * TBLK)])

    if ztail:
        pltpu.sync_copy(
            rows_v.at[pl.ds(0, ztail)],
            acc_sh.at[pl.ds(s * RPS + zmain, ztail)],
        )

    plsc.subcore_barrier()

    for ph in range(PHASES):
        base = w * TPW + ph * HPW
        pltpu.sync_copy(src_hbm.at[pl.ds(base, HPW)], idx_v.at[pl.ds(0, HPW)])
        pltpu.sync_copy(dst_hbm.at[pl.ds(base, HPW)], idx_v.at[pl.ds(HPW, HPW)])

        for t in range(NBUF - 1):
            pltpu.async_copy(g_hbm.at[idx_v.at[t]], bufs[t], sem)

        @pl.loop(0, HPW, step=NBUF)
        def _edge(j):
            for i in range(NBUF):
                buf = bufs[i]
                nbuf = bufs[(i + NBUF - 1) % NBUF]
                t = j + i
                pltpu.make_async_copy(g_hbm.at[idx_v.at[t]], buf, sem).wait()

                @pl.when(t + NBUF - 1 < HPW)
                def _next(t=t, nbuf=nbuf):
                    pltpu.async_copy(g_hbm.at[idx_v.at[t + NBUF - 1]], nbuf, sem)

                pltpu.sync_copy(buf, acc_sh.at[idx_v.at[HPW + t]], add=True)

    plsc.subcore_barrier()

    pltpu.sync_copy(
        acc_sh.at[pl.ds(s * RPS, RPS)],
        out_hbm.at[c, pl.ds(s * RPS, RPS)],
    )


def _mm_body(x_ref, w_ref, o_ref):
    o_ref[...] = lax.dot_general(
        x_ref[...], w_ref[...], (((1,), (1,)), ((), ())),
        preferred_element_type=jnp.float32,
        precision=lax.Precision.HIGHEST,
    )


def _tc_linear(x, W):
    R = 2000
    return pl.pallas_call(
        _mm_body,
        grid=(N // R,),
        in_specs=[
            pl.BlockSpec((R, D), lambda i: (i, 0)),
            pl.BlockSpec((D, D), lambda i: (0, 0)),
        ],
        out_specs=pl.BlockSpec((R, D), lambda i: (i, 0)),
        out_shape=jax.ShapeDtypeStruct((N, D), jnp.float32),
    )(x, W)


def _scale_body(hist_ref, h_ref, g_ref, d_ref):
    ones = jnp.ones((NW, 1), jnp.float32)
    deg = lax.dot_general(
        hist_ref[...], ones, (((0,), (0,)), ((), ())),
        preferred_element_type=jnp.float32,
        precision=lax.Precision.HIGHEST,
    ) + 1.0
    dis = lax.rsqrt(deg[:N])
    d_ref[...] = dis
    g_ref[...] = h_ref[...] * dis


def _tc_scale(hist, h):
    return pl.pallas_call(
        _scale_body,
        out_shape=[
            jax.ShapeDtypeStruct((N, D), jnp.float32),
            jax.ShapeDtypeStruct((N, 1), jnp.float32),
        ],
    )(hist, h)


def _epi_body(p_ref0, p_ref1, g_ref, d_ref, b_ref, o_ref):
    p0 = p_ref0[0]
    p1 = p_ref1[0]
    o_ref[...] = d_ref[...] * (p0 + p1 + g_ref[...]) + b_ref[...]


def _tc_epilogue(parts, g, dis_col, b_row):
    R = 2000
    return pl.pallas_call(
        _epi_body,
        grid=(N // R,),
        in_specs=[
            pl.BlockSpec((1, R, D), lambda i: (0, i, 0)),
            pl.BlockSpec((1, R, D), lambda i: (1, i, 0)),
            pl.BlockSpec((R, D), lambda i: (i, 0)),
            pl.BlockSpec((R, 1), lambda i: (i, 0)),
            pl.BlockSpec((1, D), lambda i: (0, 0)),
        ],
        out_specs=pl.BlockSpec((R, D), lambda i: (i, 0)),
        out_shape=jax.ShapeDtypeStruct((N, D), jnp.float32),
    )(parts, parts, g, dis_col, b_row)


def kernel(x, edge_index, W, b):
    e = edge_index.shape[1]
    src = edge_index[0].astype(jnp.int32)
    dst = edge_index[1].astype(jnp.int32)
    pad = E_PAD - e
    pad_ar = jnp.arange(pad, dtype=jnp.int32)
    src_p = jnp.concatenate([src, pad_ar % N]).reshape(NTR, TBLK)
    dst_p = jnp.concatenate([dst, DUMMY + pad_ar % (ROWS - N)]).reshape(NTR, TBLK)

    h = _tc_linear(x, W)
    hist = _sc_degree(dst_p)
    g, dis_col = _tc_scale(hist, h)
    parts = _sc_aggregate(g, src_p, dst_p)
    out = _tc_epilogue(parts, g, dis_col, b.reshape(1, D))
    return out
